# baseline (speedup 1.0000x reference)
import functools

import jax
import jax.numpy as jnp
from jax import lax
from jax.experimental import pallas as pl
from jax.experimental.pallas import tpu as pltpu
from jax.experimental.pallas import tpu_sc as plsc

N = 10000
E = 320000
NPAD = 12288
RPT = NPAD // 16
NW = 32
CH = 128
NCHUNK = 80
EPT = NCHUNK * CH
EPAD = NW * EPT
NB = 4
NGROUP = NCHUNK // NB
ZREP = RPT // CH
BR = 400
NBLK = N // BR
EPS = 1e-5


def _sc_mesh():
    return plsc.VectorSubcoreMesh(core_axis_name="c", subcore_axis_name="s")


def _agg_body(y_hbm, src_hbm, dst_hbm, zeros_hbm, out_hbm,
              src_v, dst_v, rows_v, acc_sh, g_sem, s_sem):
    c = lax.axis_index("c")
    s = lax.axis_index("s")
    wid = s * 2 + c
    pltpu.sync_copy(zeros_hbm, rows_v.at[0])
    for j in range(ZREP):
        pltpu.sync_copy(rows_v.at[0], acc_sh.at[pl.ds(s * RPT + j * CH, CH)])
    pltpu.sync_copy(src_hbm.at[wid], src_v)
    pltpu.sync_copy(dst_hbm.at[wid], dst_v)
    plsc.subcore_barrier()

    for b in range(NB):
        pltpu.async_copy(y_hbm.at[src_v.at[b]], rows_v.at[b], g_sem.at[b])

    def group(j, carry):
        for b in range(NB):
            i = j * NB + b
            pltpu.make_async_copy(y_hbm.at[src_v.at[i]], rows_v.at[b],
                                  g_sem.at[b]).wait()
            pltpu.async_copy(rows_v.at[b], acc_sh.at[dst_v.at[i]],
                             s_sem.at[b], add=True).wait()

            @pl.when(j < NGROUP - 1)
            def _():
                pltpu.async_copy(y_hbm.at[src_v.at[i + NB]], rows_v.at[b],
                                 g_sem.at[b])
        return carry

    lax.fori_loop(0, NGROUP, group, 0)
    plsc.subcore_barrier()
    pltpu.sync_copy(acc_sh.at[pl.ds(s * RPT, RPT)],
                    out_hbm.at[c, pl.ds(s * RPT, RPT)])


def _make_sc_agg(w):
    return pl.kernel(
        _agg_body,
        out_type=jax.ShapeDtypeStruct((2, NPAD, w), jnp.float32),
        mesh=_sc_mesh(),
        compiler_params=pltpu.CompilerParams(use_tc_tiling_on_sc=False),
        scratch_types=[
            pltpu.VMEM((NCHUNK, CH), jnp.int32),
            pltpu.VMEM((NCHUNK, CH), jnp.int32),
            pltpu.VMEM((NB, CH, w), jnp.float32),
            pltpu.VMEM_SHARED((NPAD, w), jnp.float32),
            pltpu.SemaphoreType.DMA((NB,)),
            pltpu.SemaphoreType.DMA((NB,)),
        ],
    )


def _agg_deg_body(y_hbm, src_hbm, dst_hbm, zeros_hbm, zeros16_hbm, ones_hbm,
                  agg_out, deg_out,
                  src_v, dst_v, rows_v, ones_v, z16_v, acc_sh, deg_sh,
                  g_sem, s_sem):
    c = lax.axis_index("c")
    s = lax.axis_index("s")
    wid = s * 2 + c
    pltpu.sync_copy(zeros_hbm, rows_v.at[0])
    pltpu.sync_copy(ones_hbm, ones_v)
    pltpu.sync_copy(zeros16_hbm, z16_v)
    for j in range(ZREP):
        pltpu.sync_copy(rows_v.at[0], acc_sh.at[pl.ds(s * RPT + j * CH, CH)])
        pltpu.sync_copy(z16_v, deg_sh.at[pl.ds(s * RPT + j * CH, CH)])
    pltpu.sync_copy(src_hbm.at[wid], src_v)
    pltpu.sync_copy(dst_hbm.at[wid], dst_v)
    plsc.subcore_barrier()

    for b in range(NB):
        pltpu.async_copy(y_hbm.at[src_v.at[b]], rows_v.at[b], g_sem.at[b])

    def group(j, carry):
        for b in range(NB):
            i = j * NB + b
            pltpu.make_async_copy(y_hbm.at[src_v.at[i]], rows_v.at[b],
                                  g_sem.at[b]).wait()
            pltpu.sync_copy(ones_v, deg_sh.at[dst_v.at[i]], add=True)
            pltpu.async_copy(rows_v.at[b], acc_sh.at[dst_v.at[i]],
                             s_sem.at[b], add=True).wait()

            @pl.when(j < NGROUP - 1)
            def _():
                pltpu.async_copy(y_hbm.at[src_v.at[i + NB]], rows_v.at[b],
                                 g_sem.at[b])
        return carry

    lax.fori_loop(0, NGROUP, group, 0)
    plsc.subcore_barrier()
    pltpu.sync_copy(acc_sh.at[pl.ds(s * RPT, RPT)],
                    agg_out.at[c, pl.ds(s * RPT, RPT)])
    pltpu.sync_copy(deg_sh.at[pl.ds(s * RPT, RPT)],
                    deg_out.at[c, pl.ds(s * RPT, RPT)])


def _make_sc_agg_deg(w):
    return pl.kernel(
        _agg_deg_body,
        out_type=(
            jax.ShapeDtypeStruct((2, NPAD, w), jnp.float32),
            jax.ShapeDtypeStruct((2, NPAD, 16), jnp.float32),
        ),
        mesh=_sc_mesh(),
        compiler_params=pltpu.CompilerParams(use_tc_tiling_on_sc=False),
        scratch_types=[
            pltpu.VMEM((NCHUNK, CH), jnp.int32),
            pltpu.VMEM((NCHUNK, CH), jnp.int32),
            pltpu.VMEM((NB, CH, w), jnp.float32),
            pltpu.VMEM((CH, 16), jnp.float32),
            pltpu.VMEM((CH, 16), jnp.float32),
            pltpu.VMEM_SHARED((NPAD, w), jnp.float32),
            pltpu.VMEM_SHARED((NPAD, 16), jnp.float32),
            pltpu.SemaphoreType.DMA((NB,)),
            pltpu.SemaphoreType.DMA((NB,)),
        ],
    )


def _mm_body(x_ref, w_ref, o_ref):
    o_ref[...] = jnp.dot(x_ref[...], w_ref[...],
                         preferred_element_type=jnp.float32)


def _matmul(x, w):
    din, dout = w.shape
    return pl.pallas_call(
        _mm_body,
        grid=(NBLK,),
        in_specs=[
            pl.BlockSpec((BR, din), lambda r: (r, 0)),
            pl.BlockSpec((din, dout), lambda r: (0, 0)),
        ],
        out_specs=pl.BlockSpec((BR, dout), lambda r: (r, 0)),
        out_shape=jax.ShapeDtypeStruct((N, dout), jnp.float32),
    )(x, w)


def _make_combine_body(npart):
    def body(*refs):
        p_refs = refs[:npart]
        d_ref, h_ref, wr_ref, bl_ref, pre_ref, stats_ref = refs[npart:]
        r = pl.program_id(0)
        halves = [p[0] + p[1] for p in p_refs]
        agg = halves[0] if npart == 1 else jnp.concatenate(halves, axis=1)
        deg = d_ref[0][:, 0:1] + d_ref[1][:, 0:1]
        aggm = agg / jnp.maximum(deg, 1.0)
        pre = aggm + bl_ref[...] + jnp.dot(h_ref[...], wr_ref[...],
                                           preferred_element_type=jnp.float32)
        pre_ref[...] = pre

        @pl.when(r == 0)
        def _():
            stats_ref[...] = jnp.zeros_like(stats_ref)

        stats_ref[0:1, :] += jnp.sum(pre, axis=0, keepdims=True)
        stats_ref[1:2, :] += jnp.sum(pre * pre, axis=0, keepdims=True)

    return body


def _combine(parts_list, deg_parts, h, wr, bl):
    din, dout = wr.shape
    wpart = parts_list[0].shape[-1]
    return pl.pallas_call(
        _make_combine_body(len(parts_list)),
        grid=(NBLK,),
        in_specs=[
            *[pl.BlockSpec((2, BR, wpart), lambda r: (0, r, 0))
              for _ in parts_list],
            pl.BlockSpec((2, BR, 16), lambda r: (0, r, 0)),
            pl.BlockSpec((BR, din), lambda r: (r, 0)),
            pl.BlockSpec((din, dout), lambda r: (0, 0)),
            pl.BlockSpec((1, dout), lambda r: (0, 0)),
        ],
        out_specs=[
            pl.BlockSpec((BR, dout), lambda r: (r, 0)),
            pl.BlockSpec((8, dout), lambda r: (0, 0)),
        ],
        out_shape=[
            jax.ShapeDtypeStruct((N, dout), jnp.float32),
            jax.ShapeDtypeStruct((8, dout), jnp.float32),
        ],
    )(*parts_list, deg_parts, h, wr, bl)


def _bn_next_body(pre_ref, stats_ref, g_ref, be_ref, wn_ref, h_ref, y_ref):
    mu = stats_ref[0:1, :] * (1.0 / N)
    var = stats_ref[1:2, :] * (1.0 / N) - mu * mu
    rstd = lax.rsqrt(var + EPS)
    h = jnp.maximum((pre_ref[...] - mu) * (rstd * g_ref[...]) + be_ref[...],
                    0.0)
    h_ref[...] = h
    y_ref[...] = jnp.dot(h, wn_ref[...], preferred_element_type=jnp.float32)


def _bn_next(pre, stats, g, be, wn):
    d, dn = wn.shape
    return pl.pallas_call(
        _bn_next_body,
        grid=(NBLK,),
        in_specs=[
            pl.BlockSpec((BR, d), lambda r: (r, 0)),
            pl.BlockSpec((8, d), lambda r: (0, 0)),
            pl.BlockSpec((1, d), lambda r: (0, 0)),
            pl.BlockSpec((1, d), lambda r: (0, 0)),
            pl.BlockSpec((d, dn), lambda r: (0, 0)),
        ],
        out_specs=[
            pl.BlockSpec((BR, d), lambda r: (r, 0)),
            pl.BlockSpec((BR, dn), lambda r: (r, 0)),
        ],
        out_shape=[
            jax.ShapeDtypeStruct((N, d), jnp.float32),
            jax.ShapeDtypeStruct((N, dn), jnp.float32),
        ],
    )(pre, stats, g, be, wn)


def _bn_head_body(pre_ref, stats_ref, g_ref, be_ref, wh_ref, bh_ref, o_ref):
    mu = stats_ref[0:1, :] * (1.0 / N)
    var = stats_ref[1:2, :] * (1.0 / N) - mu * mu
    rstd = lax.rsqrt(var + EPS)
    h = jnp.maximum((pre_ref[...] - mu) * (rstd * g_ref[...]) + be_ref[...],
                    0.0)
    o_ref[...] = jnp.dot(h, wh_ref[...],
                         preferred_element_type=jnp.float32) + bh_ref[...]


def _bn_head(pre, stats, g, be, whp, bhp):
    d = pre.shape[1]
    return pl.pallas_call(
        _bn_head_body,
        grid=(NBLK,),
        in_specs=[
            pl.BlockSpec((BR, d), lambda r: (r, 0)),
            pl.BlockSpec((8, d), lambda r: (0, 0)),
            pl.BlockSpec((1, d), lambda r: (0, 0)),
            pl.BlockSpec((1, d), lambda r: (0, 0)),
            pl.BlockSpec((d, 128), lambda r: (0, 0)),
            pl.BlockSpec((1, 128), lambda r: (0, 0)),
        ],
        out_specs=pl.BlockSpec((BR, 128), lambda r: (r, 0)),
        out_shape=jax.ShapeDtypeStruct((N, 128), jnp.float32),
    )(pre, stats, g, be, whp, bhp)


def kernel(x, edge_index, W1l, b1l, W1r, g1, be1, W2l, b2l, W2r, g2, be2,
           W3l, b3l, W3r, g3, be3, Wh, bh):
    ppw = EPT - E // NW
    pad_src = jnp.zeros((NW, ppw), jnp.int32)
    pad_dst = (N + 71 * jnp.arange(NW, dtype=jnp.int32)[:, None]
               + (jnp.arange(ppw, dtype=jnp.int32) % 71)[None, :])
    src = jnp.concatenate(
        [edge_index[0].astype(jnp.int32).reshape(NW, -1), pad_src], axis=1
    ).reshape(NW, NCHUNK, CH)
    dst = jnp.concatenate(
        [edge_index[1].astype(jnp.int32).reshape(NW, -1), pad_dst], axis=1
    ).reshape(NW, NCHUNK, CH)
    zeros64 = jnp.zeros((CH, 64), jnp.float32)
    zeros16 = jnp.zeros((CH, 16), jnp.float32)
    ones16 = jnp.ones((CH, 16), jnp.float32)
    b1l_ = b1l.reshape(1, -1)
    b2l_ = b2l.reshape(1, -1)
    b3l_ = b3l.reshape(1, -1)
    g1_, be1_ = g1.reshape(1, -1), be1.reshape(1, -1)
    g2_, be2_ = g2.reshape(1, -1), be2.reshape(1, -1)
    g3_, be3_ = g3.reshape(1, -1), be3.reshape(1, -1)
    whp = jnp.pad(Wh, ((0, 0), (0, 127)))
    bhp = jnp.pad(bh, (0, 127)).reshape(1, 128)

    y1 = _matmul(x, W1l)
    agg1a, degp = _make_sc_agg_deg(64)(y1[:, :64], src, dst, zeros64,
                                       zeros16, ones16)
    agg1b = _make_sc_agg(64)(y1[:, 64:], src, dst, zeros64)
    pre1, st1 = _combine([agg1a, agg1b], degp, x, W1r, b1l_)
    h1, y2 = _bn_next(pre1, st1, g1_, be1_, W2l)

    agg2 = _make_sc_agg(64)(y2, src, dst, zeros64)
    pre2, st2 = _combine([agg2], degp, h1, W2r, b2l_)
    h2, y3 = _bn_next(pre2, st2, g2_, be2_, W3l)

    agg3 = _make_sc_agg(64)(y3, src, dst, zeros64)
    pre3, st3 = _combine([agg3], degp, h2, W3r, b3l_)
    out = _bn_head(pre3, st3, g3_, be3_, whp, bhp)
    return out[:, 0]


# speedup vs baseline: 2.3357x; 2.3357x over previous
import functools

import jax
import jax.numpy as jnp
from jax import lax
from jax.experimental import pallas as pl
from jax.experimental.pallas import tpu as pltpu
from jax.experimental.pallas import tpu_sc as plsc

N = 10000
E = 320000
NPAD = 10240
RPT = NPAD // 16
NW = 32
CH = 80
NCHUNK = 125
EPT = NCHUNK * CH
EPAD = NW * EPT
NB = 5
NGROUP = NCHUNK // NB
ZREP = RPT // CH
BR = 400
NBLK = N // BR
EPS = 1e-5


def _sc_mesh():
    return plsc.VectorSubcoreMesh(core_axis_name="c", subcore_axis_name="s")


def _agg_body(y_hbm, src_hbm, dst_hbm, zeros_hbm, out_hbm,
              src_v, dst_v, rows_v, acc_sh, g_sem, s_sem):
    c = lax.axis_index("c")
    s = lax.axis_index("s")
    wid = s * 2 + c
    pltpu.sync_copy(zeros_hbm, rows_v.at[0])
    for j in range(ZREP):
        pltpu.sync_copy(rows_v.at[0], acc_sh.at[pl.ds(s * RPT + j * CH, CH)])
    pltpu.sync_copy(src_hbm.at[wid], src_v)
    pltpu.sync_copy(dst_hbm.at[wid], dst_v)
    plsc.subcore_barrier()

    for b in range(NB):
        pltpu.async_copy(y_hbm.at[src_v.at[b]], rows_v.at[b], g_sem.at[b])

    def group(j, carry):
        for b in range(NB):
            i = j * NB + b
            pltpu.make_async_copy(y_hbm.at[src_v.at[i]], rows_v.at[b],
                                  g_sem.at[b]).wait()
            pltpu.async_copy(rows_v.at[b], acc_sh.at[dst_v.at[i]],
                             s_sem.at[b], add=True).wait()

            @pl.when(j < NGROUP - 1)
            def _():
                pltpu.async_copy(y_hbm.at[src_v.at[i + NB]], rows_v.at[b],
                                 g_sem.at[b])
        return carry

    lax.fori_loop(0, NGROUP, group, 0)
    plsc.subcore_barrier()
    pltpu.sync_copy(acc_sh.at[pl.ds(s * RPT, RPT)],
                    out_hbm.at[c, pl.ds(s * RPT, RPT)])


def _make_sc_agg(w):
    return pl.kernel(
        _agg_body,
        out_type=jax.ShapeDtypeStruct((2, NPAD, w), jnp.float32),
        mesh=_sc_mesh(),
        compiler_params=pltpu.CompilerParams(use_tc_tiling_on_sc=False),
        scratch_types=[
            pltpu.VMEM((NCHUNK, CH), jnp.int32),
            pltpu.VMEM((NCHUNK, CH), jnp.int32),
            pltpu.VMEM((NB, CH, w), jnp.float32),
            pltpu.VMEM_SHARED((NPAD, w), jnp.float32),
            pltpu.SemaphoreType.DMA((NB,)),
            pltpu.SemaphoreType.DMA((NB,)),
        ],
    )


def _agg_deg_body(y_hbm, src_hbm, dst_hbm, zeros_hbm, zeros16_hbm, ones_hbm,
                  agg_out, deg_out,
                  src_v, dst_v, rows_v, ones_v, z16_v, acc_sh, deg_sh,
                  g_sem, s_sem):
    c = lax.axis_index("c")
    s = lax.axis_index("s")
    wid = s * 2 + c
    pltpu.sync_copy(zeros_hbm, rows_v.at[0])
    pltpu.sync_copy(ones_hbm, ones_v)
    pltpu.sync_copy(zeros16_hbm, z16_v)
    for j in range(ZREP):
        pltpu.sync_copy(rows_v.at[0], acc_sh.at[pl.ds(s * RPT + j * CH, CH)])
        pltpu.sync_copy(z16_v, deg_sh.at[pl.ds(s * RPT + j * CH, CH)])
    pltpu.sync_copy(src_hbm.at[wid], src_v)
    pltpu.sync_copy(dst_hbm.at[wid], dst_v)
    plsc.subcore_barrier()

    for b in range(NB):
        pltpu.async_copy(y_hbm.at[src_v.at[b]], rows_v.at[b], g_sem.at[b])

    def group(j, carry):
        for b in range(NB):
            i = j * NB + b
            pltpu.make_async_copy(y_hbm.at[src_v.at[i]], rows_v.at[b],
                                  g_sem.at[b]).wait()
            pltpu.sync_copy(ones_v, deg_sh.at[dst_v.at[i]], add=True)
            pltpu.async_copy(rows_v.at[b], acc_sh.at[dst_v.at[i]],
                             s_sem.at[b], add=True).wait()

            @pl.when(j < NGROUP - 1)
            def _():
                pltpu.async_copy(y_hbm.at[src_v.at[i + NB]], rows_v.at[b],
                                 g_sem.at[b])
        return carry

    lax.fori_loop(0, NGROUP, group, 0)
    plsc.subcore_barrier()
    pltpu.sync_copy(acc_sh.at[pl.ds(s * RPT, RPT)],
                    agg_out.at[c, pl.ds(s * RPT, RPT)])
    pltpu.sync_copy(deg_sh.at[pl.ds(s * RPT, RPT)],
                    deg_out.at[c, pl.ds(s * RPT, RPT)])


def _make_sc_agg_deg(w):
    return pl.kernel(
        _agg_deg_body,
        out_type=(
            jax.ShapeDtypeStruct((2, NPAD, w), jnp.float32),
            jax.ShapeDtypeStruct((2, NPAD, 16), jnp.float32),
        ),
        mesh=_sc_mesh(),
        compiler_params=pltpu.CompilerParams(use_tc_tiling_on_sc=False),
        scratch_types=[
            pltpu.VMEM((NCHUNK, CH), jnp.int32),
            pltpu.VMEM((NCHUNK, CH), jnp.int32),
            pltpu.VMEM((NB, CH, w), jnp.float32),
            pltpu.VMEM((CH, 16), jnp.float32),
            pltpu.VMEM((CH, 16), jnp.float32),
            pltpu.VMEM_SHARED((NPAD, w), jnp.float32),
            pltpu.VMEM_SHARED((NPAD, 16), jnp.float32),
            pltpu.SemaphoreType.DMA((NB,)),
            pltpu.SemaphoreType.DMA((NB,)),
        ],
    )


def _mm_body(x_ref, w_ref, o_ref):
    o_ref[...] = jnp.dot(x_ref[...], w_ref[...],
                         preferred_element_type=jnp.float32)


def _matmul(x, w):
    din, dout = w.shape
    return pl.pallas_call(
        _mm_body,
        grid=(NBLK,),
        in_specs=[
            pl.BlockSpec((BR, din), lambda r: (r, 0)),
            pl.BlockSpec((din, dout), lambda r: (0, 0)),
        ],
        out_specs=pl.BlockSpec((BR, dout), lambda r: (r, 0)),
        out_shape=jax.ShapeDtypeStruct((N, dout), jnp.float32),
    )(x, w)


def _make_combine_body(npart):
    def body(*refs):
        p_refs = refs[:npart]
        d_ref, h_ref, wr_ref, bl_ref, pre_ref, stats_ref = refs[npart:]
        r = pl.program_id(0)
        halves = [p[0] + p[1] for p in p_refs]
        agg = halves[0] if npart == 1 else jnp.concatenate(halves, axis=1)
        deg = d_ref[0][:, 0:1] + d_ref[1][:, 0:1]
        aggm = agg / jnp.maximum(deg, 1.0)
        pre = aggm + bl_ref[...] + jnp.dot(h_ref[...], wr_ref[...],
                                           preferred_element_type=jnp.float32)
        pre_ref[...] = pre

        @pl.when(r == 0)
        def _():
            stats_ref[...] = jnp.zeros_like(stats_ref)

        stats_ref[0:1, :] += jnp.sum(pre, axis=0, keepdims=True)
        stats_ref[1:2, :] += jnp.sum(pre * pre, axis=0, keepdims=True)

    return body


def _combine(parts_list, deg_parts, h, wr, bl):
    din, dout = wr.shape
    wpart = parts_list[0].shape[-1]
    return pl.pallas_call(
        _make_combine_body(len(parts_list)),
        grid=(NBLK,),
        in_specs=[
            *[pl.BlockSpec((2, BR, wpart), lambda r: (0, r, 0))
              for _ in parts_list],
            pl.BlockSpec((2, BR, 16), lambda r: (0, r, 0)),
            pl.BlockSpec((BR, din), lambda r: (r, 0)),
            pl.BlockSpec((din, dout), lambda r: (0, 0)),
            pl.BlockSpec((1, dout), lambda r: (0, 0)),
        ],
        out_specs=[
            pl.BlockSpec((BR, dout), lambda r: (r, 0)),
            pl.BlockSpec((8, dout), lambda r: (0, 0)),
        ],
        out_shape=[
            jax.ShapeDtypeStruct((N, dout), jnp.float32),
            jax.ShapeDtypeStruct((8, dout), jnp.float32),
        ],
    )(*parts_list, deg_parts, h, wr, bl)


def _bn_next_body(pre_ref, stats_ref, g_ref, be_ref, wn_ref, h_ref, y_ref):
    mu = stats_ref[0:1, :] * (1.0 / N)
    var = stats_ref[1:2, :] * (1.0 / N) - mu * mu
    rstd = lax.rsqrt(var + EPS)
    h = jnp.maximum((pre_ref[...] - mu) * (rstd * g_ref[...]) + be_ref[...],
                    0.0)
    h_ref[...] = h
    y_ref[...] = jnp.dot(h, wn_ref[...], preferred_element_type=jnp.float32)


def _bn_next(pre, stats, g, be, wn):
    d, dn = wn.shape
    return pl.pallas_call(
        _bn_next_body,
        grid=(NBLK,),
        in_specs=[
            pl.BlockSpec((BR, d), lambda r: (r, 0)),
            pl.BlockSpec((8, d), lambda r: (0, 0)),
            pl.BlockSpec((1, d), lambda r: (0, 0)),
            pl.BlockSpec((1, d), lambda r: (0, 0)),
            pl.BlockSpec((d, dn), lambda r: (0, 0)),
        ],
        out_specs=[
            pl.BlockSpec((BR, d), lambda r: (r, 0)),
            pl.BlockSpec((BR, dn), lambda r: (r, 0)),
        ],
        out_shape=[
            jax.ShapeDtypeStruct((N, d), jnp.float32),
            jax.ShapeDtypeStruct((N, dn), jnp.float32),
        ],
    )(pre, stats, g, be, wn)


def _bn_head_body(pre_ref, stats_ref, g_ref, be_ref, wh_ref, bh_ref, o_ref):
    mu = stats_ref[0:1, :] * (1.0 / N)
    var = stats_ref[1:2, :] * (1.0 / N) - mu * mu
    rstd = lax.rsqrt(var + EPS)
    h = jnp.maximum((pre_ref[...] - mu) * (rstd * g_ref[...]) + be_ref[...],
                    0.0)
    o_ref[...] = jnp.dot(h, wh_ref[...],
                         preferred_element_type=jnp.float32) + bh_ref[...]


def _bn_head(pre, stats, g, be, whp, bhp):
    d = pre.shape[1]
    return pl.pallas_call(
        _bn_head_body,
        grid=(NBLK,),
        in_specs=[
            pl.BlockSpec((BR, d), lambda r: (r, 0)),
            pl.BlockSpec((8, d), lambda r: (0, 0)),
            pl.BlockSpec((1, d), lambda r: (0, 0)),
            pl.BlockSpec((1, d), lambda r: (0, 0)),
            pl.BlockSpec((d, 128), lambda r: (0, 0)),
            pl.BlockSpec((1, 128), lambda r: (0, 0)),
        ],
        out_specs=pl.BlockSpec((BR, 128), lambda r: (r, 0)),
        out_shape=jax.ShapeDtypeStruct((N, 128), jnp.float32),
    )(pre, stats, g, be, whp, bhp)


def kernel(x, edge_index, W1l, b1l, W1r, g1, be1, W2l, b2l, W2r, g2, be2,
           W3l, b3l, W3r, g3, be3, Wh, bh):
    src = edge_index[0].astype(jnp.int32).reshape(NW, NCHUNK, CH)
    dst = edge_index[1].astype(jnp.int32).reshape(NW, NCHUNK, CH)
    zeros64 = jnp.zeros((CH, 64), jnp.float32)
    zeros16 = jnp.zeros((CH, 16), jnp.float32)
    ones16 = jnp.ones((CH, 16), jnp.float32)
    b1l_ = b1l.reshape(1, -1)
    b2l_ = b2l.reshape(1, -1)
    b3l_ = b3l.reshape(1, -1)
    g1_, be1_ = g1.reshape(1, -1), be1.reshape(1, -1)
    g2_, be2_ = g2.reshape(1, -1), be2.reshape(1, -1)
    g3_, be3_ = g3.reshape(1, -1), be3.reshape(1, -1)
    whp = jnp.pad(Wh, ((0, 0), (0, 127)))
    bhp = jnp.pad(bh, (0, 127)).reshape(1, 128)

    y1 = _matmul(x, W1l)
    agg1a, degp = _make_sc_agg_deg(64)(y1[:, :64], src, dst, zeros64,
                                       zeros16, ones16)
    agg1b = _make_sc_agg(64)(y1[:, 64:], src, dst, zeros64)
    pre1, st1 = _combine([agg1a, agg1b], degp, x, W1r, b1l_)
    h1, y2 = _bn_next(pre1, st1, g1_, be1_, W2l)

    agg2 = _make_sc_agg(64)(y2, src, dst, zeros64)
    pre2, st2 = _combine([agg2], degp, h1, W2r, b2l_)
    h2, y3 = _bn_next(pre2, st2, g2_, be2_, W3l)

    agg3 = _make_sc_agg(64)(y3, src, dst, zeros64)
    pre3, st3 = _combine([agg3], degp, h2, W3r, b3l_)
    out = _bn_head(pre3, st3, g3_, be3_, whp, bhp)
    return out[:, 0]


# speedup vs baseline: 2.4633x; 1.0546x over previous
import functools

import jax
import jax.numpy as jnp
from jax import lax
from jax.experimental import pallas as pl
from jax.experimental.pallas import tpu as pltpu
from jax.experimental.pallas import tpu_sc as plsc

N = 10000
E = 320000
NPAD = 10240
RPT = NPAD // 16
NW = 32
CH = 80
NCHUNK = 125
EPT = NCHUNK * CH
EPAD = NW * EPT
NB = 5
NGROUP = NCHUNK // NB
ZREP = RPT // CH
BR = 400
NBLK = N // BR
EPS = 1e-5


def _sc_mesh():
    return plsc.VectorSubcoreMesh(core_axis_name="c", subcore_axis_name="s")


def _agg_body(y_hbm, src_hbm, dst_hbm, zeros_hbm, out_hbm,
              src_v, dst_v, rows_v, acc_sh, g_sem, s_sem):
    c = lax.axis_index("c")
    s = lax.axis_index("s")
    wid = s * 2 + c
    pltpu.sync_copy(zeros_hbm, rows_v.at[0])
    for j in range(ZREP):
        pltpu.sync_copy(rows_v.at[0], acc_sh.at[pl.ds(s * RPT + j * CH, CH)])
    pltpu.sync_copy(src_hbm.at[wid], src_v)
    pltpu.sync_copy(dst_hbm.at[wid], dst_v)
    plsc.subcore_barrier()

    for b in range(NB):
        pltpu.async_copy(y_hbm.at[src_v.at[b]], rows_v.at[b], g_sem.at[b])

    def group(j, carry):
        for b in range(NB):
            i = j * NB + b
            pltpu.make_async_copy(y_hbm.at[src_v.at[i]], rows_v.at[b],
                                  g_sem.at[b]).wait()
            pltpu.async_copy(rows_v.at[b], acc_sh.at[dst_v.at[i]],
                             s_sem.at[b], add=True).wait()

            @pl.when(j < NGROUP - 1)
            def _():
                pltpu.async_copy(y_hbm.at[src_v.at[i + NB]], rows_v.at[b],
                                 g_sem.at[b])
        return carry

    lax.fori_loop(0, NGROUP, group, 0)
    plsc.subcore_barrier()
    pltpu.sync_copy(acc_sh.at[pl.ds(s * RPT, RPT)],
                    out_hbm.at[c, pl.ds(s * RPT, RPT)])


def _make_sc_agg(w):
    return pl.kernel(
        _agg_body,
        out_type=jax.ShapeDtypeStruct((2, NPAD, w), jnp.float32),
        mesh=_sc_mesh(),
        compiler_params=pltpu.CompilerParams(use_tc_tiling_on_sc=False),
        scratch_types=[
            pltpu.VMEM((NCHUNK, CH), jnp.int32),
            pltpu.VMEM((NCHUNK, CH), jnp.int32),
            pltpu.VMEM((NB, CH, w), jnp.float32),
            pltpu.VMEM_SHARED((NPAD, w), jnp.float32),
            pltpu.SemaphoreType.DMA((NB,)),
            pltpu.SemaphoreType.DMA((NB,)),
        ],
    )


def _agg_deg_body(y_hbm, src_hbm, dst_hbm, zeros_hbm, zeros16_hbm, ones_hbm,
                  agg_out, deg_out,
                  src_v, dst_v, rows_v, ones_v, z16_v, acc_sh, deg_sh,
                  g_sem, s_sem):
    c = lax.axis_index("c")
    s = lax.axis_index("s")
    wid = s * 2 + c
    pltpu.sync_copy(zeros_hbm, rows_v.at[0])
    pltpu.sync_copy(ones_hbm, ones_v)
    pltpu.sync_copy(zeros16_hbm, z16_v)
    for j in range(ZREP):
        pltpu.sync_copy(rows_v.at[0], acc_sh.at[pl.ds(s * RPT + j * CH, CH)])
        pltpu.sync_copy(z16_v, deg_sh.at[pl.ds(s * RPT + j * CH, CH)])
    pltpu.sync_copy(src_hbm.at[wid], src_v)
    pltpu.sync_copy(dst_hbm.at[wid], dst_v)
    plsc.subcore_barrier()

    for b in range(NB):
        pltpu.async_copy(y_hbm.at[src_v.at[b]], rows_v.at[b], g_sem.at[b])

    def group(j, carry):
        for b in range(NB):
            i = j * NB + b
            pltpu.make_async_copy(y_hbm.at[src_v.at[i]], rows_v.at[b],
                                  g_sem.at[b]).wait()
            pltpu.sync_copy(ones_v, deg_sh.at[dst_v.at[i]], add=True)
            pltpu.async_copy(rows_v.at[b], acc_sh.at[dst_v.at[i]],
                             s_sem.at[b], add=True).wait()

            @pl.when(j < NGROUP - 1)
            def _():
                pltpu.async_copy(y_hbm.at[src_v.at[i + NB]], rows_v.at[b],
                                 g_sem.at[b])
        return carry

    lax.fori_loop(0, NGROUP, group, 0)
    plsc.subcore_barrier()
    pltpu.sync_copy(acc_sh.at[pl.ds(s * RPT, RPT)],
                    agg_out.at[c, pl.ds(s * RPT, RPT)])
    pltpu.sync_copy(deg_sh.at[pl.ds(s * RPT, RPT)],
                    deg_out.at[c, pl.ds(s * RPT, RPT)])


def _make_sc_agg_deg(w):
    return pl.kernel(
        _agg_deg_body,
        out_type=(
            jax.ShapeDtypeStruct((2, NPAD, w), jnp.float32),
            jax.ShapeDtypeStruct((2, NPAD, 16), jnp.float32),
        ),
        mesh=_sc_mesh(),
        compiler_params=pltpu.CompilerParams(use_tc_tiling_on_sc=False),
        scratch_types=[
            pltpu.VMEM((NCHUNK, CH), jnp.int32),
            pltpu.VMEM((NCHUNK, CH), jnp.int32),
            pltpu.VMEM((NB, CH, w), jnp.float32),
            pltpu.VMEM((CH, 16), jnp.float32),
            pltpu.VMEM((CH, 16), jnp.float32),
            pltpu.VMEM_SHARED((NPAD, w), jnp.float32),
            pltpu.VMEM_SHARED((NPAD, 16), jnp.float32),
            pltpu.SemaphoreType.DMA((NB,)),
            pltpu.SemaphoreType.DMA((NB,)),
        ],
    )


def _mm_body(x_ref, w_ref, o_ref):
    o_ref[...] = jnp.dot(x_ref[...], w_ref[...],
                         preferred_element_type=jnp.float32)


def _matmul(x, w):
    din, dout = w.shape
    return pl.pallas_call(
        _mm_body,
        grid=(NBLK,),
        in_specs=[
            pl.BlockSpec((BR, din), lambda r: (r, 0)),
            pl.BlockSpec((din, dout), lambda r: (0, 0)),
        ],
        out_specs=pl.BlockSpec((BR, dout), lambda r: (r, 0)),
        out_shape=jax.ShapeDtypeStruct((N, dout), jnp.float32),
    )(x, w)


def _make_layer_body(npart, head):
    def body(*refs):
        p_refs = refs[:npart]
        d_ref, h_ref, wr_ref, bl_ref, g_ref, be_ref, wn_ref, bn_ref = \
            refs[npart:npart + 8]
        outs = refs[npart + 8:-2]
        pre_s, stats_s = refs[-2:]
        p = pl.program_id(0)
        r = pl.program_id(1)

        @pl.when(p == 0)
        def _():
            halves = [q[0] + q[1] for q in p_refs]
            agg = halves[0] if npart == 1 else jnp.concatenate(halves, axis=1)
            deg = d_ref[0][:, 0:1] + d_ref[1][:, 0:1]
            aggm = agg / jnp.maximum(deg, 1.0)
            pre = aggm + bl_ref[...] + jnp.dot(
                h_ref[...], wr_ref[...], preferred_element_type=jnp.float32)
            pre_s[pl.ds(r * BR, BR), :] = pre

            @pl.when(r == 0)
            def _():
                stats_s[...] = jnp.zeros_like(stats_s)

            stats_s[0:1, :] += jnp.sum(pre, axis=0, keepdims=True)
            stats_s[1:2, :] += jnp.sum(pre * pre, axis=0, keepdims=True)

        @pl.when(p == 1)
        def _():
            mu = stats_s[0:1, :] * (1.0 / N)
            var = stats_s[1:2, :] * (1.0 / N) - mu * mu
            rstd = lax.rsqrt(var + EPS)
            pre = pre_s[pl.ds(r * BR, BR), :]
            h = jnp.maximum((pre - mu) * (rstd * g_ref[...]) + be_ref[...],
                            0.0)
            if head:
                outs[0][...] = jnp.dot(
                    h, wn_ref[...],
                    preferred_element_type=jnp.float32) + bn_ref[...]
            else:
                outs[0][...] = h
                outs[1][...] = jnp.dot(h, wn_ref[...],
                                       preferred_element_type=jnp.float32)

    return body


def _layer(parts_list, deg_parts, h, wr, bl, g, be, wn, bn, head=False):
    din, dout = wr.shape
    dn = wn.shape[1]
    wpart = parts_list[0].shape[-1]
    if head:
        out_specs = [pl.BlockSpec((BR, dn), lambda p, r: (r * p, 0))]
        out_shape = [jax.ShapeDtypeStruct((N, dn), jnp.float32)]
    else:
        out_specs = [
            pl.BlockSpec((BR, dout), lambda p, r: (r * p, 0)),
            pl.BlockSpec((BR, dn), lambda p, r: (r * p, 0)),
        ]
        out_shape = [
            jax.ShapeDtypeStruct((N, dout), jnp.float32),
            jax.ShapeDtypeStruct((N, dn), jnp.float32),
        ]
    return pl.pallas_call(
        _make_layer_body(len(parts_list), head),
        grid=(2, NBLK),
        in_specs=[
            *[pl.BlockSpec((2, BR, wpart), lambda p, r: (0, r * (1 - p), 0))
              for _ in parts_list],
            pl.BlockSpec((2, BR, 16), lambda p, r: (0, r * (1 - p), 0)),
            pl.BlockSpec((BR, din), lambda p, r: (r * (1 - p), 0)),
            pl.BlockSpec((din, dout), lambda p, r: (0, 0)),
            pl.BlockSpec((1, dout), lambda p, r: (0, 0)),
            pl.BlockSpec((1, dout), lambda p, r: (0, 0)),
            pl.BlockSpec((1, dout), lambda p, r: (0, 0)),
            pl.BlockSpec((dout, dn), lambda p, r: (0, 0)),
            pl.BlockSpec((1, dn), lambda p, r: (0, 0)),
        ],
        out_specs=out_specs,
        out_shape=out_shape,
        scratch_shapes=[
            pltpu.VMEM((N, dout), jnp.float32),
            pltpu.VMEM((8, dout), jnp.float32),
        ],
    )(*parts_list, deg_parts, h, wr, bl, g, be, wn, bn)


def kernel(x, edge_index, W1l, b1l, W1r, g1, be1, W2l, b2l, W2r, g2, be2,
           W3l, b3l, W3r, g3, be3, Wh, bh):
    src = edge_index[0].astype(jnp.int32).reshape(NW, NCHUNK, CH)
    dst = edge_index[1].astype(jnp.int32).reshape(NW, NCHUNK, CH)
    zeros64 = jnp.zeros((CH, 64), jnp.float32)
    zeros16 = jnp.zeros((CH, 16), jnp.float32)
    ones16 = jnp.ones((CH, 16), jnp.float32)
    b1l_ = b1l.reshape(1, -1)
    b2l_ = b2l.reshape(1, -1)
    b3l_ = b3l.reshape(1, -1)
    g1_, be1_ = g1.reshape(1, -1), be1.reshape(1, -1)
    g2_, be2_ = g2.reshape(1, -1), be2.reshape(1, -1)
    g3_, be3_ = g3.reshape(1, -1), be3.reshape(1, -1)
    whp = jnp.pad(Wh, ((0, 0), (0, 127)))
    bhp = jnp.pad(bh, (0, 127)).reshape(1, 128)

    y1 = _matmul(x, W1l)
    agg1a, degp = _make_sc_agg_deg(64)(y1[:, :64], src, dst, zeros64,
                                       zeros16, ones16)
    agg1b = _make_sc_agg(64)(y1[:, 64:], src, dst, zeros64)
    h1, y2 = _layer([agg1a, agg1b], degp, x, W1r, b1l_, g1_, be1_, W2l,
                    b1l_[:, :64])

    agg2 = _make_sc_agg(64)(y2, src, dst, zeros64)
    h2, y3 = _layer([agg2], degp, h1, W2r, b2l_, g2_, be2_, W3l,
                    b2l_)

    agg3 = _make_sc_agg(64)(y3, src, dst, zeros64)
    (out,) = _layer([agg3], degp, h2, W3r, b3l_, g3_, be3_, whp, bhp,
                    head=True)
    return out[:, 0]


# speedup vs baseline: 2.8048x; 1.1386x over previous
import functools

import jax
import jax.numpy as jnp
from jax import lax
from jax.experimental import pallas as pl
from jax.experimental.pallas import tpu as pltpu
from jax.experimental.pallas import tpu_sc as plsc

N = 10000
E = 320000
NPAD = 10240
RPT = NPAD // 16
NW = 32
CH = 80
NCHUNK = 125
EPT = NCHUNK * CH
EPAD = NW * EPT
NBUF = 10
HB = NBUF // 2
NGROUP = NCHUNK // NBUF
NTAIL = NCHUNK - NGROUP * NBUF
ZREP = RPT // CH
BR = 2000
NBLK = N // BR
EPS = 1e-5


def _sc_mesh():
    return plsc.VectorSubcoreMesh(core_axis_name="c", subcore_axis_name="s")


def _edge_pipeline(y_hbm, src_v, dst_v, rows_v, acc_sh, g_sem, s_sem,
                   extra_scatter=None):
    def wait_gather(i, b):
        pltpu.make_async_copy(y_hbm.at[src_v.at[i]], rows_v.at[b],
                              g_sem.at[b]).wait()

    def issue_scatter(i, b):
        pltpu.async_copy(rows_v.at[b], acc_sh.at[dst_v.at[i]],
                         s_sem.at[b], add=True)
        if extra_scatter is not None:
            extra_scatter(i, b)

    def wait_scatter(i, b):
        pltpu.make_async_copy(rows_v.at[b], acc_sh.at[dst_v.at[i]],
                              s_sem.at[b]).wait()
        if extra_scatter is not None:
            extra_scatter(i, b, wait=True)

    for b in range(HB):
        pltpu.async_copy(y_hbm.at[src_v.at[b]], rows_v.at[b], g_sem.at[b])

    def group(j, carry):
        base = j * NBUF
        for b in range(NBUF):
            i = base + b
            wait_gather(i, b)
            issue_scatter(i, b)
            b2 = (b + HB) % NBUF
            if b < HB:
                @pl.when(j > 0)
                def _():
                    wait_scatter(i - HB, b2)
            else:
                wait_scatter(i - HB, b2)
            pltpu.async_copy(y_hbm.at[src_v.at[i + HB]], rows_v.at[b2],
                             g_sem.at[b2])
        return carry

    lax.fori_loop(0, NGROUP, group, 0)
    base = NGROUP * NBUF
    for b in range(NTAIL):
        i = base + b
        wait_gather(i, b)
        issue_scatter(i, b)
    for b in range(NTAIL, NBUF):
        wait_scatter(base - NBUF + b, b)
    for b in range(NTAIL):
        wait_scatter(base + b, b)


def _agg_body(y_hbm, src_hbm, dst_hbm, zeros_hbm, out_hbm,
              src_v, dst_v, rows_v, acc_sh, g_sem, s_sem):
    c = lax.axis_index("c")
    s = lax.axis_index("s")
    wid = s * 2 + c
    pltpu.sync_copy(zeros_hbm, rows_v.at[0])
    for j in range(ZREP):
        pltpu.sync_copy(rows_v.at[0], acc_sh.at[pl.ds(s * RPT + j * CH, CH)])
    pltpu.sync_copy(src_hbm.at[wid], src_v)
    pltpu.sync_copy(dst_hbm.at[wid], dst_v)
    plsc.subcore_barrier()
    _edge_pipeline(y_hbm, src_v, dst_v, rows_v, acc_sh, g_sem, s_sem)
    plsc.subcore_barrier()
    pltpu.sync_copy(acc_sh.at[pl.ds(s * RPT, RPT)],
                    out_hbm.at[c, pl.ds(s * RPT, RPT)])


def _make_sc_agg(w):
    return pl.kernel(
        _agg_body,
        out_type=jax.ShapeDtypeStruct((2, NPAD, w), jnp.float32),
        mesh=_sc_mesh(),
        compiler_params=pltpu.CompilerParams(use_tc_tiling_on_sc=False),
        scratch_types=[
            pltpu.VMEM((NCHUNK, CH), jnp.int32),
            pltpu.VMEM((NCHUNK, CH), jnp.int32),
            pltpu.VMEM((NBUF, CH, w), jnp.float32),
            pltpu.VMEM_SHARED((NPAD, w), jnp.float32),
            pltpu.SemaphoreType.DMA((NBUF,)),
            pltpu.SemaphoreType.DMA((NBUF,)),
        ],
    )


def _agg_deg_body(y_hbm, src_hbm, dst_hbm, zeros_hbm, zeros16_hbm, ones_hbm,
                  agg_out, deg_out,
                  src_v, dst_v, rows_v, ones_v, z16_v, acc_sh, deg_sh,
                  g_sem, s_sem):
    c = lax.axis_index("c")
    s = lax.axis_index("s")
    wid = s * 2 + c
    pltpu.sync_copy(zeros_hbm, rows_v.at[0])
    pltpu.sync_copy(ones_hbm, ones_v)
    pltpu.sync_copy(zeros16_hbm, z16_v)
    for j in range(ZREP):
        pltpu.sync_copy(rows_v.at[0], acc_sh.at[pl.ds(s * RPT + j * CH, CH)])
        pltpu.sync_copy(z16_v, deg_sh.at[pl.ds(s * RPT + j * CH, CH)])
    pltpu.sync_copy(src_hbm.at[wid], src_v)
    pltpu.sync_copy(dst_hbm.at[wid], dst_v)
    plsc.subcore_barrier()

    def deg_scatter(i, b, wait=False):
        if wait:
            pltpu.make_async_copy(ones_v, deg_sh.at[dst_v.at[i]],
                                  s_sem.at[b]).wait()
        else:
            pltpu.async_copy(ones_v, deg_sh.at[dst_v.at[i]], s_sem.at[b],
                             add=True)

    _edge_pipeline(y_hbm, src_v, dst_v, rows_v, acc_sh, g_sem, s_sem,
                   extra_scatter=deg_scatter)
    plsc.subcore_barrier()
    pltpu.sync_copy(acc_sh.at[pl.ds(s * RPT, RPT)],
                    agg_out.at[c, pl.ds(s * RPT, RPT)])
    pltpu.sync_copy(deg_sh.at[pl.ds(s * RPT, RPT)],
                    deg_out.at[c, pl.ds(s * RPT, RPT)])


def _make_sc_agg_deg(w):
    return pl.kernel(
        _agg_deg_body,
        out_type=(
            jax.ShapeDtypeStruct((2, NPAD, w), jnp.float32),
            jax.ShapeDtypeStruct((2, NPAD, 16), jnp.float32),
        ),
        mesh=_sc_mesh(),
        compiler_params=pltpu.CompilerParams(use_tc_tiling_on_sc=False),
        scratch_types=[
            pltpu.VMEM((NCHUNK, CH), jnp.int32),
            pltpu.VMEM((NCHUNK, CH), jnp.int32),
            pltpu.VMEM((NBUF, CH, w), jnp.float32),
            pltpu.VMEM((CH, 16), jnp.float32),
            pltpu.VMEM((CH, 16), jnp.float32),
            pltpu.VMEM_SHARED((NPAD, w), jnp.float32),
            pltpu.VMEM_SHARED((NPAD, 16), jnp.float32),
            pltpu.SemaphoreType.DMA((NBUF,)),
            pltpu.SemaphoreType.DMA((NBUF,)),
        ],
    )


def _mm_split_body(x_ref, w_ref, oa_ref, ob_ref):
    y = jnp.dot(x_ref[...], w_ref[...], preferred_element_type=jnp.float32)
    oa_ref[...] = y[:, :64]
    ob_ref[...] = y[:, 64:]


def _matmul_split(x, w):
    return pl.pallas_call(
        _mm_split_body,
        grid=(NBLK,),
        in_specs=[
            pl.BlockSpec((BR, 128), lambda r: (r, 0)),
            pl.BlockSpec((128, 128), lambda r: (0, 0)),
        ],
        out_specs=[
            pl.BlockSpec((BR, 64), lambda r: (r, 0)),
            pl.BlockSpec((BR, 64), lambda r: (r, 0)),
        ],
        out_shape=[
            jax.ShapeDtypeStruct((N, 64), jnp.float32),
            jax.ShapeDtypeStruct((N, 64), jnp.float32),
        ],
    )(x, w)


def _make_layer_body(npart, head):
    def body(*refs):
        p_refs = refs[:npart]
        d_ref, h_ref, wr_ref, bl_ref, g_ref, be_ref, wn_ref, bn_ref = \
            refs[npart:npart + 8]
        outs = refs[npart + 8:-2]
        pre_s, stats_s = refs[-2:]
        p = pl.program_id(0)
        r = pl.program_id(1)

        @pl.when(p == 0)
        def _():
            halves = [q[0] + q[1] for q in p_refs]
            agg = halves[0] if npart == 1 else jnp.concatenate(halves, axis=1)
            deg = d_ref[0][:, 0:1] + d_ref[1][:, 0:1]
            aggm = agg / jnp.maximum(deg, 1.0)
            pre = aggm + bl_ref[...] + jnp.dot(
                h_ref[...], wr_ref[...], preferred_element_type=jnp.float32)
            pre_s[pl.ds(r * BR, BR), :] = pre

            @pl.when(r == 0)
            def _():
                stats_s[...] = jnp.zeros_like(stats_s)

            stats_s[0:1, :] += jnp.sum(pre, axis=0, keepdims=True)
            stats_s[1:2, :] += jnp.sum(pre * pre, axis=0, keepdims=True)

        @pl.when(p == 1)
        def _():
            mu = stats_s[0:1, :] * (1.0 / N)
            var = stats_s[1:2, :] * (1.0 / N) - mu * mu
            rstd = lax.rsqrt(var + EPS)
            pre = pre_s[pl.ds(r * BR, BR), :]
            h = jnp.maximum((pre - mu) * (rstd * g_ref[...]) + be_ref[...],
                            0.0)
            if head:
                outs[0][...] = jnp.dot(
                    h, wn_ref[...],
                    preferred_element_type=jnp.float32) + bn_ref[...]
            else:
                outs[0][...] = h
                outs[1][...] = jnp.dot(h, wn_ref[...],
                                       preferred_element_type=jnp.float32)

    return body


def _layer(parts_list, deg_parts, h, wr, bl, g, be, wn, bn, head=False):
    din, dout = wr.shape
    dn = wn.shape[1]
    wpart = parts_list[0].shape[-1]
    if head:
        out_specs = [pl.BlockSpec((BR, dn), lambda p, r: (r * p, 0))]
        out_shape = [jax.ShapeDtypeStruct((N, dn), jnp.float32)]
    else:
        out_specs = [
            pl.BlockSpec((BR, dout), lambda p, r: (r * p, 0)),
            pl.BlockSpec((BR, dn), lambda p, r: (r * p, 0)),
        ]
        out_shape = [
            jax.ShapeDtypeStruct((N, dout), jnp.float32),
            jax.ShapeDtypeStruct((N, dn), jnp.float32),
        ]
    return pl.pallas_call(
        _make_layer_body(len(parts_list), head),
        grid=(2, NBLK),
        in_specs=[
            *[pl.BlockSpec((2, BR, wpart), lambda p, r: (0, r * (1 - p), 0))
              for _ in parts_list],
            pl.BlockSpec((2, BR, 16), lambda p, r: (0, r * (1 - p), 0)),
            pl.BlockSpec((BR, din), lambda p, r: (r * (1 - p), 0)),
            pl.BlockSpec((din, dout), lambda p, r: (0, 0)),
            pl.BlockSpec((1, dout), lambda p, r: (0, 0)),
            pl.BlockSpec((1, dout), lambda p, r: (0, 0)),
            pl.BlockSpec((1, dout), lambda p, r: (0, 0)),
            pl.BlockSpec((dout, dn), lambda p, r: (0, 0)),
            pl.BlockSpec((1, dn), lambda p, r: (0, 0)),
        ],
        out_specs=out_specs,
        out_shape=out_shape,
        scratch_shapes=[
            pltpu.VMEM((N, dout), jnp.float32),
            pltpu.VMEM((8, dout), jnp.float32),
        ],
    )(*parts_list, deg_parts, h, wr, bl, g, be, wn, bn)


def kernel(x, edge_index, W1l, b1l, W1r, g1, be1, W2l, b2l, W2r, g2, be2,
           W3l, b3l, W3r, g3, be3, Wh, bh):
    src = edge_index[0].astype(jnp.int32).reshape(NW, NCHUNK, CH)
    dst = edge_index[1].astype(jnp.int32).reshape(NW, NCHUNK, CH)
    zeros64 = jnp.zeros((CH, 64), jnp.float32)
    zeros16 = jnp.zeros((CH, 16), jnp.float32)
    ones16 = jnp.ones((CH, 16), jnp.float32)
    b1l_ = b1l.reshape(1, -1)
    b2l_ = b2l.reshape(1, -1)
    b3l_ = b3l.reshape(1, -1)
    g1_, be1_ = g1.reshape(1, -1), be1.reshape(1, -1)
    g2_, be2_ = g2.reshape(1, -1), be2.reshape(1, -1)
    g3_, be3_ = g3.reshape(1, -1), be3.reshape(1, -1)
    whp = jnp.pad(Wh, ((0, 0), (0, 127)))
    bhp = jnp.pad(bh, (0, 127)).reshape(1, 128)

    y1a, y1b = _matmul_split(x, W1l)
    agg1a, degp = _make_sc_agg_deg(64)(y1a, src, dst, zeros64,
                                       zeros16, ones16)
    agg1b = _make_sc_agg(64)(y1b, src, dst, zeros64)
    h1, y2 = _layer([agg1a, agg1b], degp, x, W1r, b1l_, g1_, be1_, W2l,
                    b1l_[:, :64])

    agg2 = _make_sc_agg(64)(y2, src, dst, zeros64)
    h2, y3 = _layer([agg2], degp, h1, W2r, b2l_, g2_, be2_, W3l,
                    b2l_)

    agg3 = _make_sc_agg(64)(y3, src, dst, zeros64)
    (out,) = _layer([agg3], degp, h2, W3r, b3l_, g3_, be3_, whp, bhp,
                    head=True)
    return out[:, 0]
